# baseline (device time: 9316 ns/iter reference)
import jax
import jax.numpy as jnp
from jax import lax
from jax.experimental import pallas as pl
from jax.experimental.pallas import tpu as pltpu

NBLK = 8


def kernel(x):
    m, n_local = x.shape
    n_global = 2 * n_local
    bm = m // NBLK
    rows = bm // 128
    half = (NBLK // 2) * rows

    def body(x_hbm, out_ref, xbuf, send_ref, recv_ref,
             copy_sems, send_sems, recv_sems):
        my_x = lax.axis_index("x")
        my_y = lax.axis_index("y")
        nbr = (my_x, 1 - my_y)
        barrier_sem = pltpu.get_barrier_semaphore()
        pl.semaphore_signal(
            barrier_sem, inc=1, device_id=nbr,
            device_id_type=pl.DeviceIdType.MESH,
        )

        def half_rdma(h):
            return pltpu.make_async_remote_copy(
                src_ref=send_ref.at[pl.ds(h * half, half)],
                dst_ref=recv_ref.at[pl.ds(h * half, half)],
                send_sem=send_sems.at[h],
                recv_sem=recv_sems.at[h],
                device_id=nbr,
                device_id_type=pl.DeviceIdType.MESH,
            )

        copies = [
            pltpu.make_async_copy(
                x_hbm.at[pl.ds(i * bm, bm), :], xbuf.at[i], copy_sems.at[i]
            )
            for i in range(NBLK)
        ]
        for c in copies:
            c.start()

        for i in range(NBLK):
            copies[i].wait()
            partial = jnp.sum(
                xbuf[i], axis=1, keepdims=True, dtype=jnp.float32
            )
            send_ref[pl.ds(i * rows, rows), :] = partial.reshape(rows, 128)
            if i == NBLK // 2 - 1:
                pl.semaphore_wait(barrier_sem, 1)
                half_rdma(0).start()

        half_rdma(1).start()
        half_rdma(0).wait()
        half_rdma(1).wait()
        out_ref[...] = (send_ref[...] + recv_ref[...]) * (1.0 / n_global)

    packed = pl.pallas_call(
        body,
        out_shape=jax.ShapeDtypeStruct((m // 128, 128), jnp.float32),
        in_specs=[pl.BlockSpec(memory_space=pl.ANY)],
        out_specs=pl.BlockSpec(memory_space=pltpu.VMEM),
        scratch_shapes=[
            pltpu.VMEM((NBLK, bm, n_local), jnp.float32),
            pltpu.VMEM((m // 128, 128), jnp.float32),
            pltpu.VMEM((m // 128, 128), jnp.float32),
            pltpu.SemaphoreType.DMA((NBLK,)),
            pltpu.SemaphoreType.DMA((2,)),
            pltpu.SemaphoreType.DMA((2,)),
        ],
        compiler_params=pltpu.CompilerParams(collective_id=0),
    )(x)
    return jnp.reshape(packed, (m, 1))


# device time: 8309 ns/iter; 1.1212x vs baseline; 1.1212x over previous
import jax
import jax.numpy as jnp
from jax import lax
from jax.experimental import pallas as pl
from jax.experimental.pallas import tpu as pltpu

GRID = 2


def kernel(x):
    m, n_local = x.shape
    n_global = 2 * n_local
    bm = m // GRID
    rows = bm // 128
    half = (GRID // 2) * rows

    def body(x_ref, out_ref, send_ref, recv_ref, send_sems, recv_sems):
        pi = pl.program_id(0)
        my_x = lax.axis_index("x")
        my_y = lax.axis_index("y")
        nbr = (my_x, 1 - my_y)
        barrier_sem = pltpu.get_barrier_semaphore()

        def half_rdma(h):
            return pltpu.make_async_remote_copy(
                src_ref=send_ref.at[pl.ds(h * half, half)],
                dst_ref=recv_ref.at[pl.ds(h * half, half)],
                send_sem=send_sems.at[h],
                recv_sem=recv_sems.at[h],
                device_id=nbr,
                device_id_type=pl.DeviceIdType.MESH,
            )

        @pl.when(pi == 0)
        def _():
            pl.semaphore_signal(
                barrier_sem, inc=1, device_id=nbr,
                device_id_type=pl.DeviceIdType.MESH,
            )

        partial = jnp.sum(x_ref[...], axis=1, keepdims=True, dtype=jnp.float32)
        send_ref[pl.ds(pi * rows, rows), :] = partial.reshape(rows, 128)

        @pl.when(pi == GRID // 2 - 1)
        def _():
            pl.semaphore_wait(barrier_sem, 1)
            half_rdma(0).start()

        @pl.when(pi == GRID - 1)
        def _():
            half_rdma(1).start()
            half_rdma(0).wait()
            half_rdma(1).wait()
            out_ref[...] = (send_ref[...] + recv_ref[...]) * (1.0 / n_global)

    packed = pl.pallas_call(
        body,
        grid=(GRID,),
        out_shape=jax.ShapeDtypeStruct((m // 128, 128), jnp.float32),
        in_specs=[
            pl.BlockSpec((bm, n_local), lambda i: (i, 0)),
        ],
        out_specs=pl.BlockSpec((m // 128, 128), lambda i: (0, 0)),
        scratch_shapes=[
            pltpu.VMEM((m // 128, 128), jnp.float32),
            pltpu.VMEM((m // 128, 128), jnp.float32),
            pltpu.SemaphoreType.DMA((2,)),
            pltpu.SemaphoreType.DMA((2,)),
        ],
        compiler_params=pltpu.CompilerParams(collective_id=0),
    )(x)
    return jnp.reshape(packed, (m, 1))
